# baseline (speedup 1.0000x reference)
import functools

import jax
import jax.numpy as jnp
from jax import lax
from jax.experimental import pallas as pl
from jax.experimental.pallas import tpu as pltpu
from jax.experimental.pallas import tpu_sc as plsc

N = 10000
E = 320000
D = 128
KT = 2
NLAYERS = 2

NC = 2
NS = 16
NW = NC * NS
CH = 112
NCHUNK = 90
EPADN = NW * NCHUNK * CH - E
NPAD = 10240
RPT = NPAD // NS

BN = 1000
NB = N // BN

_sc_mesh = plsc.VectorSubcoreMesh(
    core_axis_name="c", subcore_axis_name="s", num_cores=NC, num_subcores=NS
)


def _mm_body(x_ref, w_ref, o_ref):
    o_ref[...] = jnp.dot(x_ref[...], w_ref[0], preferred_element_type=jnp.float32)


def _project(x, Wt):
    return pl.pallas_call(
        _mm_body,
        grid=(KT, NB),
        in_specs=[
            pl.BlockSpec((BN, D), lambda k, i: (i, 0)),
            pl.BlockSpec((1, D, D), lambda k, i: (k, 0, 0)),
        ],
        out_specs=pl.BlockSpec((BN, D), lambda k, i: (k * NB + i, 0)),
        out_shape=jax.ShapeDtypeStruct((KT * N, D), jnp.float32),
    )(x, Wt)


@functools.partial(
    pl.kernel,
    out_type=jax.ShapeDtypeStruct((NC, NPAD, D), jnp.float32),
    mesh=_sc_mesh,
    scratch_types=[
        pltpu.VMEM((2, CH), jnp.int32),
        pltpu.VMEM((2, CH), jnp.int32),
        pltpu.VMEM((2, CH), jnp.int32),
        pltpu.VMEM((CH, D), jnp.float32),
        pltpu.VMEM((CH, D), jnp.float32),
        pltpu.VMEM((CH, D), jnp.float32),
        pltpu.VMEM_SHARED((NPAD, D), jnp.float32),
        pltpu.SemaphoreType.DMA,
        pltpu.SemaphoreType.DMA,
        pltpu.SemaphoreType.DMA,
        pltpu.SemaphoreType.DMA,
        pltpu.SemaphoreType.DMA,
        pltpu.SemaphoreType.DMA,
    ],
)
def _sc_edge(h_hbm, eidx_hbm, zer_hbm, out_hbm,
             ib0, ib1, ib2, rows0, rows1, rows2, acc_sh,
             isem0, isem1, isem2, gsem0, gsem1, gsem2):
    cid = lax.axis_index("c")
    sid = lax.axis_index("s")
    wid = cid * NS + sid
    my_eidx = eidx_hbm.at[wid]
    ib = (ib0, ib1, ib2)
    rows = (rows0, rows1, rows2)
    isem = (isem0, isem1, isem2)
    gsem = (gsem0, gsem1, gsem2)

    pltpu.sync_copy(zer_hbm, acc_sh.at[pl.ds(sid * RPT, RPT)])
    plsc.subcore_barrier()

    pltpu.sync_copy(my_eidx.at[0], ib0)
    pltpu.sync_copy(my_eidx.at[1], ib1)
    pltpu.async_copy(my_eidx.at[2], ib2, isem2)
    pltpu.async_copy(h_hbm.at[ib0.at[0]], rows0, gsem0)
    pltpu.async_copy(h_hbm.at[ib1.at[0]], rows1, gsem1)

    def step(c, a):
        b = (a + 2) % 3

        @pl.when(c + 2 < NCHUNK)
        def _fire_gather():
            pltpu.make_async_copy(my_eidx.at[c + 2], ib[b], isem[b]).wait()
            pltpu.async_copy(h_hbm.at[ib[b].at[0]], rows[b], gsem[b])

        pltpu.make_async_copy(h_hbm.at[ib[a].at[0]], rows[a], gsem[a]).wait()
        pltpu.sync_copy(rows[a], acc_sh.at[ib[a].at[1]], add=True)

        @pl.when(c + 3 < NCHUNK)
        def _prefetch_idx():
            pltpu.async_copy(my_eidx.at[c + 3], ib[a], isem[a])

    def body(c, carry):
        for a in range(3):
            @pl.when(c % 3 == a)
            def _(a=a):
                step(c, a)
        return carry

    lax.fori_loop(0, NCHUNK, body, 0)
    plsc.subcore_barrier()

    pltpu.sync_copy(acc_sh.at[pl.ds(sid * RPT, RPT)],
                    out_hbm.at[cid].at[pl.ds(sid * RPT, RPT)])


def _comb_body(x_ref, p_ref, o_ref):
    s = p_ref[0] + p_ref[1]
    y = x_ref[...] + jnp.maximum(s, 0.0)
    nrm = jnp.sqrt(jnp.sum(y * y, axis=1, keepdims=True))
    o_ref[...] = y / jnp.maximum(nrm, 1e-12)


def _combine(x, part):
    return pl.pallas_call(
        _comb_body,
        grid=(NB,),
        in_specs=[
            pl.BlockSpec((BN, D), lambda i: (i, 0)),
            pl.BlockSpec((NC, BN, D), lambda i: (0, i, 0)),
        ],
        out_specs=pl.BlockSpec((BN, D), lambda i: (i, 0)),
        out_shape=jax.ShapeDtypeStruct((N, D), jnp.float32),
    )(x, part)


def kernel(x, edge_index, edge_attr, W):
    src = edge_index[0]
    dst = edge_index[1]
    gidx = (edge_attr - 1) * N + src
    ppt = EPADN // NW
    pad_g = jnp.broadcast_to(
        (jnp.arange(ppt, dtype=jnp.int32) * 128) % (KT * N), (NW, ppt)
    )
    pad_d = jnp.broadcast_to(
        N + (jnp.arange(ppt, dtype=jnp.int32) % (NPAD - N)), (NW, ppt)
    )
    gidxp = jnp.concatenate([gidx.reshape(NW, E // NW), pad_g], axis=1)
    dstp = jnp.concatenate([dst.reshape(NW, E // NW), pad_d], axis=1)
    eidx = jnp.stack(
        [gidxp.reshape(NW, NCHUNK, CH), dstp.reshape(NW, NCHUNK, CH)], axis=2
    )
    zer = jnp.zeros((RPT, D), jnp.float32)
    for t in range(NLAYERS):
        h = _project(x, W[t])
        part = _sc_edge(h, eidx, zer)
        x = _combine(x, part)
    return x


# speedup vs baseline: 1.0198x; 1.0198x over previous
import functools

import jax
import jax.numpy as jnp
from jax import lax
from jax.experimental import pallas as pl
from jax.experimental.pallas import tpu as pltpu
from jax.experimental.pallas import tpu_sc as plsc

N = 10000
E = 320000
D = 128
KT = 2
NLAYERS = 2

NC = 2
NS = 16
NW = NC * NS
CH = 128
NCHUNK = 79
EPADN = NW * NCHUNK * CH - E
NPAD = 10240
RPT = NPAD // NS

BN = 1000
NB = N // BN

_sc_mesh = plsc.VectorSubcoreMesh(
    core_axis_name="c", subcore_axis_name="s", num_cores=NC, num_subcores=NS
)


def _mm_body(x_ref, w_ref, o_ref):
    o_ref[...] = jnp.dot(x_ref[...], w_ref[0], preferred_element_type=jnp.float32)


def _project(x, Wt):
    return pl.pallas_call(
        _mm_body,
        grid=(KT, NB),
        in_specs=[
            pl.BlockSpec((BN, D), lambda k, i: (i, 0)),
            pl.BlockSpec((1, D, D), lambda k, i: (k, 0, 0)),
        ],
        out_specs=pl.BlockSpec((BN, D), lambda k, i: (k * NB + i, 0)),
        out_shape=jax.ShapeDtypeStruct((KT * N, D), jnp.float32),
    )(x, Wt)


@functools.partial(
    pl.kernel,
    out_type=jax.ShapeDtypeStruct((NC, NPAD, D), jnp.float32),
    mesh=_sc_mesh,
    scratch_types=[
        pltpu.VMEM((2, CH), jnp.int32),
        pltpu.VMEM((2, CH), jnp.int32),
        pltpu.VMEM((CH, D), jnp.float32),
        pltpu.VMEM((CH, D), jnp.float32),
        pltpu.VMEM_SHARED((NPAD, D), jnp.float32),
        pltpu.SemaphoreType.DMA,
        pltpu.SemaphoreType.DMA,
        pltpu.SemaphoreType.DMA,
        pltpu.SemaphoreType.DMA,
    ],
)
def _sc_edge(h_hbm, eidx_hbm, zer_hbm, out_hbm,
             ib0, ib1, rows0, rows1, acc_sh, isem0, isem1, gsem0, gsem1):
    cid = lax.axis_index("c")
    sid = lax.axis_index("s")
    wid = cid * NS + sid
    my_eidx = eidx_hbm.at[wid]
    ib = (ib0, ib1)
    rows = (rows0, rows1)
    isem = (isem0, isem1)
    gsem = (gsem0, gsem1)

    pltpu.sync_copy(zer_hbm, acc_sh.at[pl.ds(sid * RPT, RPT)])
    plsc.subcore_barrier()

    pltpu.sync_copy(my_eidx.at[0], ib0)
    pltpu.async_copy(h_hbm.at[ib0.at[0]], rows0, gsem0)
    pltpu.async_copy(my_eidx.at[1], ib1, isem1)

    def step(c, a):
        b = (a + 1) % 2

        @pl.when(c + 1 < NCHUNK)
        def _fire_gather():
            pltpu.make_async_copy(my_eidx.at[c + 1], ib[b], isem[b]).wait()
            pltpu.async_copy(h_hbm.at[ib[b].at[0]], rows[b], gsem[b])

        pltpu.make_async_copy(h_hbm.at[ib[a].at[0]], rows[a], gsem[a]).wait()
        pltpu.sync_copy(rows[a], acc_sh.at[ib[a].at[1]], add=True)

        @pl.when(c + 2 < NCHUNK)
        def _prefetch_idx():
            pltpu.async_copy(my_eidx.at[c + 2], ib[a], isem[a])

    def body(c, carry):
        for a in range(2):
            @pl.when(c % 2 == a)
            def _(a=a):
                step(c, a)
        return carry

    lax.fori_loop(0, NCHUNK, body, 0)
    plsc.subcore_barrier()

    pltpu.sync_copy(acc_sh.at[pl.ds(sid * RPT, RPT)],
                    out_hbm.at[cid].at[pl.ds(sid * RPT, RPT)])


def _comb_body(x_ref, p_ref, o_ref):
    s = p_ref[0] + p_ref[1]
    y = x_ref[...] + jnp.maximum(s, 0.0)
    nrm = jnp.sqrt(jnp.sum(y * y, axis=1, keepdims=True))
    o_ref[...] = y / jnp.maximum(nrm, 1e-12)


def _combine(x, part):
    return pl.pallas_call(
        _comb_body,
        grid=(NB,),
        in_specs=[
            pl.BlockSpec((BN, D), lambda i: (i, 0)),
            pl.BlockSpec((NC, BN, D), lambda i: (0, i, 0)),
        ],
        out_specs=pl.BlockSpec((BN, D), lambda i: (i, 0)),
        out_shape=jax.ShapeDtypeStruct((N, D), jnp.float32),
    )(x, part)


def _cp_body(x_ref, p_ref, w_ref, x1_ref, h_ref):
    s = p_ref[0] + p_ref[1]
    y = x_ref[...] + jnp.maximum(s, 0.0)
    nrm = jnp.sqrt(jnp.sum(y * y, axis=1, keepdims=True))
    x1 = y / jnp.maximum(nrm, 1e-12)
    x1_ref[...] = x1
    h_ref[...] = jnp.dot(x1, w_ref[0], preferred_element_type=jnp.float32)


def _combine_project(x, part, Wt):
    return pl.pallas_call(
        _cp_body,
        grid=(KT, NB),
        in_specs=[
            pl.BlockSpec((BN, D), lambda k, i: (i, 0)),
            pl.BlockSpec((NC, BN, D), lambda k, i: (0, i, 0)),
            pl.BlockSpec((1, D, D), lambda k, i: (k, 0, 0)),
        ],
        out_specs=[
            pl.BlockSpec((BN, D), lambda k, i: (i, 0)),
            pl.BlockSpec((BN, D), lambda k, i: (k * NB + i, 0)),
        ],
        out_shape=[
            jax.ShapeDtypeStruct((N, D), jnp.float32),
            jax.ShapeDtypeStruct((KT * N, D), jnp.float32),
        ],
    )(x, part, Wt)


def kernel(x, edge_index, edge_attr, W):
    src = edge_index[0]
    dst = edge_index[1]
    gidx = (edge_attr - 1) * N + src
    ppt = EPADN // NW
    pad_g = jnp.broadcast_to(
        (jnp.arange(ppt, dtype=jnp.int32) * 128) % (KT * N), (NW, ppt)
    )
    pad_d = jnp.broadcast_to(
        N + (jnp.arange(ppt, dtype=jnp.int32) % (NPAD - N)), (NW, ppt)
    )
    gidxp = jnp.concatenate([gidx.reshape(NW, E // NW), pad_g], axis=1)
    dstp = jnp.concatenate([dst.reshape(NW, E // NW), pad_d], axis=1)
    eidx = jnp.stack(
        [gidxp.reshape(NW, NCHUNK, CH), dstp.reshape(NW, NCHUNK, CH)], axis=2
    )
    zer = jnp.zeros((RPT, D), jnp.float32)
    h = _project(x, W[0])
    part = _sc_edge(h, eidx, zer)
    x, h = _combine_project(x, part, W[1])
    part = _sc_edge(h, eidx, zer)
    return _combine(x, part)


# speedup vs baseline: 1.0374x; 1.0173x over previous
import functools

import jax
import jax.numpy as jnp
from jax import lax
from jax.experimental import pallas as pl
from jax.experimental.pallas import tpu as pltpu
from jax.experimental.pallas import tpu_sc as plsc

N = 10000
E = 320000
D = 128
KT = 2
NLAYERS = 2

NC = 2
NS = 16
NW = NC * NS
CH = 128
NCHUNK = 79
EPADN = NW * NCHUNK * CH - E
NPAD = 10240
RPT = NPAD // NS

BN = 1000
NB = N // BN

_sc_mesh = plsc.VectorSubcoreMesh(
    core_axis_name="c", subcore_axis_name="s", num_cores=NC, num_subcores=NS
)


def _mm_body(x_ref, w_ref, o_ref):
    o_ref[...] = jnp.dot(x_ref[...], w_ref[0], preferred_element_type=jnp.float32)


def _project(x, Wt):
    return pl.pallas_call(
        _mm_body,
        grid=(KT, NB),
        in_specs=[
            pl.BlockSpec((BN, D), lambda k, i: (i, 0)),
            pl.BlockSpec((1, D, D), lambda k, i: (k, 0, 0)),
        ],
        out_specs=pl.BlockSpec((BN, D), lambda k, i: (k * NB + i, 0)),
        out_shape=jax.ShapeDtypeStruct((KT * N, D), jnp.float32),
    )(x, Wt)


@functools.partial(
    pl.kernel,
    out_type=jax.ShapeDtypeStruct((NC, NPAD, D), jnp.float32),
    mesh=_sc_mesh,
    scratch_types=[
        pltpu.VMEM((2, CH), jnp.int32),
        pltpu.VMEM((2, CH), jnp.int32),
        pltpu.VMEM((CH, D), jnp.float32),
        pltpu.VMEM((CH, D), jnp.float32),
        pltpu.VMEM((64, D), jnp.float32),
        pltpu.VMEM_SHARED((NPAD, D), jnp.float32),
        pltpu.SemaphoreType.DMA,
        pltpu.SemaphoreType.DMA,
        pltpu.SemaphoreType.DMA,
        pltpu.SemaphoreType.DMA,
        pltpu.SemaphoreType.DMA,
    ],
)
def _sc_edge(h_hbm, eidx_hbm, zer_hbm, out_hbm,
             ib0, ib1, rows0, rows1, zbuf, acc_sh,
             isem0, isem1, gsem0, gsem1, zsem):
    cid = lax.axis_index("c")
    sid = lax.axis_index("s")
    wid = cid * NS + sid
    my_eidx = eidx_hbm.at[wid]
    ib = (ib0, ib1)
    rows = (rows0, rows1)
    isem = (isem0, isem1)
    gsem = (gsem0, gsem1)

    pltpu.sync_copy(my_eidx.at[0], ib0)
    pltpu.async_copy(h_hbm.at[ib0.at[0]], rows0, gsem0)
    pltpu.async_copy(my_eidx.at[1], ib1, isem1)

    pltpu.sync_copy(zer_hbm, zbuf)
    for p in range(RPT // 64):
        pltpu.async_copy(zbuf, acc_sh.at[pl.ds(sid * RPT + p * 64, 64)], zsem)
    for p in range(RPT // 64):
        pltpu.make_async_copy(zbuf, acc_sh.at[pl.ds(sid * RPT + p * 64, 64)],
                              zsem).wait()
    plsc.subcore_barrier()

    def step(c, a):
        b = (a + 1) % 2

        @pl.when(c + 1 < NCHUNK)
        def _fire_gather():
            pltpu.make_async_copy(my_eidx.at[c + 1], ib[b], isem[b]).wait()
            pltpu.async_copy(h_hbm.at[ib[b].at[0]], rows[b], gsem[b])

        pltpu.make_async_copy(h_hbm.at[ib[a].at[0]], rows[a], gsem[a]).wait()
        pltpu.sync_copy(rows[a], acc_sh.at[ib[a].at[1]], add=True)

        @pl.when(c + 2 < NCHUNK)
        def _prefetch_idx():
            pltpu.async_copy(my_eidx.at[c + 2], ib[a], isem[a])

    def body(c, carry):
        for a in range(2):
            @pl.when(c % 2 == a)
            def _(a=a):
                step(c, a)
        return carry

    lax.fori_loop(0, NCHUNK, body, 0)
    plsc.subcore_barrier()

    pltpu.sync_copy(acc_sh.at[pl.ds(sid * RPT, RPT)],
                    out_hbm.at[cid].at[pl.ds(sid * RPT, RPT)])


def _comb_body(x_ref, p_ref, o_ref):
    s = p_ref[0] + p_ref[1]
    y = x_ref[...] + jnp.maximum(s, 0.0)
    nrm = jnp.sqrt(jnp.sum(y * y, axis=1, keepdims=True))
    o_ref[...] = y / jnp.maximum(nrm, 1e-12)


def _combine(x, part):
    return pl.pallas_call(
        _comb_body,
        grid=(NB,),
        in_specs=[
            pl.BlockSpec((BN, D), lambda i: (i, 0)),
            pl.BlockSpec((NC, BN, D), lambda i: (0, i, 0)),
        ],
        out_specs=pl.BlockSpec((BN, D), lambda i: (i, 0)),
        out_shape=jax.ShapeDtypeStruct((N, D), jnp.float32),
    )(x, part)


def _cp_body(x_ref, p_ref, w_ref, x1_ref, h_ref):
    s = p_ref[0] + p_ref[1]
    y = x_ref[...] + jnp.maximum(s, 0.0)
    nrm = jnp.sqrt(jnp.sum(y * y, axis=1, keepdims=True))
    x1 = y / jnp.maximum(nrm, 1e-12)
    x1_ref[...] = x1
    h_ref[...] = jnp.dot(x1, w_ref[0], preferred_element_type=jnp.float32)


def _combine_project(x, part, Wt):
    return pl.pallas_call(
        _cp_body,
        grid=(KT, NB),
        in_specs=[
            pl.BlockSpec((BN, D), lambda k, i: (i, 0)),
            pl.BlockSpec((NC, BN, D), lambda k, i: (0, i, 0)),
            pl.BlockSpec((1, D, D), lambda k, i: (k, 0, 0)),
        ],
        out_specs=[
            pl.BlockSpec((BN, D), lambda k, i: (i, 0)),
            pl.BlockSpec((BN, D), lambda k, i: (k * NB + i, 0)),
        ],
        out_shape=[
            jax.ShapeDtypeStruct((N, D), jnp.float32),
            jax.ShapeDtypeStruct((KT * N, D), jnp.float32),
        ],
    )(x, part, Wt)


def kernel(x, edge_index, edge_attr, W):
    src = edge_index[0]
    dst = edge_index[1]
    gidx = (edge_attr - 1) * N + src
    ppt = EPADN // NW
    pad_g = jnp.broadcast_to(
        (jnp.arange(ppt, dtype=jnp.int32) * 128) % (KT * N), (NW, ppt)
    )
    pad_d = jnp.broadcast_to(
        N + (jnp.arange(ppt, dtype=jnp.int32) % (NPAD - N)), (NW, ppt)
    )
    gidxp = jnp.concatenate([gidx.reshape(NW, E // NW), pad_g], axis=1)
    dstp = jnp.concatenate([dst.reshape(NW, E // NW), pad_d], axis=1)
    eidx = jnp.stack(
        [gidxp.reshape(NW, NCHUNK, CH), dstp.reshape(NW, NCHUNK, CH)], axis=2
    )
    zer = jnp.zeros((64, D), jnp.float32)
    h = _project(x, W[0])
    part = _sc_edge(h, eidx, zer)
    x, h = _combine_project(x, part, W[1])
    part = _sc_edge(h, eidx, zer)
    return _combine(x, part)


# speedup vs baseline: 1.1446x; 1.1033x over previous
import functools

import jax
import jax.numpy as jnp
from jax import lax
from jax.experimental import pallas as pl
from jax.experimental.pallas import tpu as pltpu
from jax.experimental.pallas import tpu_sc as plsc

N = 10000
E = 320000
D = 128
KT = 2
NLAYERS = 2

NC = 2
NS = 16
NW = NC * NS
CH = 128
NCHUNK = 79
EPADN = NW * NCHUNK * CH - E
NPAD = 10240
RPT = NPAD // NS

BN = 1000
NB = N // BN

_sc_mesh = plsc.VectorSubcoreMesh(
    core_axis_name="c", subcore_axis_name="s", num_cores=NC, num_subcores=NS
)


def _mm_body(x_ref, w_ref, o_ref):
    o_ref[...] = jnp.dot(x_ref[...], w_ref[0], preferred_element_type=jnp.float32)


def _project(x, Wt):
    return pl.pallas_call(
        _mm_body,
        grid=(KT, NB),
        in_specs=[
            pl.BlockSpec((BN, D), lambda k, i: (i, 0)),
            pl.BlockSpec((1, D, D), lambda k, i: (k, 0, 0)),
        ],
        out_specs=pl.BlockSpec((BN, D), lambda k, i: (k * NB + i, 0)),
        out_shape=jax.ShapeDtypeStruct((KT * N, D), jnp.float32),
    )(x, Wt)


@functools.partial(
    pl.kernel,
    out_type=jax.ShapeDtypeStruct((NC, NPAD, D), jnp.float32),
    mesh=_sc_mesh,
    scratch_types=[
        pltpu.VMEM((2, CH), jnp.int32),
        pltpu.VMEM((2, CH), jnp.int32),
        pltpu.VMEM((2, CH), jnp.int32),
        pltpu.VMEM((2, CH), jnp.int32),
        pltpu.VMEM((CH, D), jnp.float32),
        pltpu.VMEM((CH, D), jnp.float32),
        pltpu.VMEM((64, D), jnp.float32),
        pltpu.VMEM_SHARED((NPAD, D), jnp.float32),
        pltpu.SemaphoreType.DMA,
        pltpu.SemaphoreType.DMA,
        pltpu.SemaphoreType.DMA,
        pltpu.SemaphoreType.DMA,
        pltpu.SemaphoreType.DMA,
        pltpu.SemaphoreType.DMA,
        pltpu.SemaphoreType.DMA,
        pltpu.SemaphoreType.DMA,
        pltpu.SemaphoreType.DMA,
    ],
)
def _sc_edge(h_hbm, eidx_hbm, zer_hbm, out_hbm,
             ib0, ib1, ib2, ib3, rows0, rows1, zbuf, acc_sh,
             isem0, isem1, isem2, isem3, gsem0, gsem1, ssem0, ssem1, zsem):
    cid = lax.axis_index("c")
    sid = lax.axis_index("s")
    wid = cid * NS + sid
    my_eidx = eidx_hbm.at[wid]
    ib = (ib0, ib1, ib2, ib3)
    rows = (rows0, rows1)
    isem = (isem0, isem1, isem2, isem3)
    gsem = (gsem0, gsem1)
    ssem = (ssem0, ssem1)

    pltpu.sync_copy(my_eidx.at[0], ib0)
    pltpu.async_copy(my_eidx.at[1], ib1, isem1)
    pltpu.async_copy(h_hbm.at[ib0.at[0]], rows0, gsem0)

    pltpu.sync_copy(zer_hbm, zbuf)
    for p in range(RPT // 64):
        pltpu.async_copy(zbuf, acc_sh.at[pl.ds(sid * RPT + p * 64, 64)], zsem)
    for p in range(RPT // 64):
        pltpu.make_async_copy(zbuf, acc_sh.at[pl.ds(sid * RPT + p * 64, 64)],
                              zsem).wait()
    plsc.subcore_barrier()

    def step(c, a, i):
        b = (a + 1) % 2

        @pl.when(c + 1 < NCHUNK)
        def _fire_gather():
            pltpu.make_async_copy(my_eidx.at[c + 1], ib[(i + 1) % 4],
                                  isem[(i + 1) % 4]).wait()

            @pl.when(c >= 1)
            def _rows_free():
                pltpu.make_async_copy(rows[b], acc_sh.at[ib[0].at[1]],
                                      ssem[b]).wait()

            pltpu.async_copy(h_hbm.at[ib[(i + 1) % 4].at[0]], rows[b], gsem[b])

        pltpu.make_async_copy(h_hbm.at[ib[i].at[0]], rows[a], gsem[a]).wait()
        pltpu.async_copy(rows[a], acc_sh.at[ib[i].at[1]], ssem[a], add=True)

        @pl.when(c + 2 < NCHUNK)
        def _prefetch_idx():
            pltpu.async_copy(my_eidx.at[c + 2], ib[(i + 2) % 4],
                             isem[(i + 2) % 4])

    def body(c, carry):
        for k in range(4):
            @pl.when(c % 4 == k)
            def _(k=k):
                step(c, k % 2, k)
        return carry

    lax.fori_loop(0, NCHUNK, body, 0)
    pltpu.make_async_copy(rows0, acc_sh.at[ib0.at[1]], ssem0).wait()
    pltpu.make_async_copy(rows1, acc_sh.at[ib1.at[1]], ssem1).wait()
    plsc.subcore_barrier()

    pltpu.sync_copy(acc_sh.at[pl.ds(sid * RPT, RPT)],
                    out_hbm.at[cid].at[pl.ds(sid * RPT, RPT)])


def _comb_body(x_ref, p_ref, o_ref):
    s = p_ref[0] + p_ref[1]
    y = x_ref[...] + jnp.maximum(s, 0.0)
    nrm = jnp.sqrt(jnp.sum(y * y, axis=1, keepdims=True))
    o_ref[...] = y / jnp.maximum(nrm, 1e-12)


def _combine(x, part):
    return pl.pallas_call(
        _comb_body,
        grid=(NB,),
        in_specs=[
            pl.BlockSpec((BN, D), lambda i: (i, 0)),
            pl.BlockSpec((NC, BN, D), lambda i: (0, i, 0)),
        ],
        out_specs=pl.BlockSpec((BN, D), lambda i: (i, 0)),
        out_shape=jax.ShapeDtypeStruct((N, D), jnp.float32),
    )(x, part)


def _cp_body(x_ref, p_ref, w_ref, x1_ref, h_ref):
    s = p_ref[0] + p_ref[1]
    y = x_ref[...] + jnp.maximum(s, 0.0)
    nrm = jnp.sqrt(jnp.sum(y * y, axis=1, keepdims=True))
    x1 = y / jnp.maximum(nrm, 1e-12)
    x1_ref[...] = x1
    h_ref[...] = jnp.dot(x1, w_ref[0], preferred_element_type=jnp.float32)


def _combine_project(x, part, Wt):
    return pl.pallas_call(
        _cp_body,
        grid=(KT, NB),
        in_specs=[
            pl.BlockSpec((BN, D), lambda k, i: (i, 0)),
            pl.BlockSpec((NC, BN, D), lambda k, i: (0, i, 0)),
            pl.BlockSpec((1, D, D), lambda k, i: (k, 0, 0)),
        ],
        out_specs=[
            pl.BlockSpec((BN, D), lambda k, i: (i, 0)),
            pl.BlockSpec((BN, D), lambda k, i: (k * NB + i, 0)),
        ],
        out_shape=[
            jax.ShapeDtypeStruct((N, D), jnp.float32),
            jax.ShapeDtypeStruct((KT * N, D), jnp.float32),
        ],
    )(x, part, Wt)


def kernel(x, edge_index, edge_attr, W):
    src = edge_index[0]
    dst = edge_index[1]
    gidx = (edge_attr - 1) * N + src
    ppt = EPADN // NW
    pad_g = jnp.broadcast_to(
        (jnp.arange(ppt, dtype=jnp.int32) * 128) % (KT * N), (NW, ppt)
    )
    pad_d = jnp.broadcast_to(
        N + (jnp.arange(ppt, dtype=jnp.int32) % (NPAD - N)), (NW, ppt)
    )
    gidxp = jnp.concatenate([gidx.reshape(NW, E // NW), pad_g], axis=1)
    dstp = jnp.concatenate([dst.reshape(NW, E // NW), pad_d], axis=1)
    eidx = jnp.stack(
        [gidxp.reshape(NW, NCHUNK, CH), dstp.reshape(NW, NCHUNK, CH)], axis=2
    )
    zer = jnp.zeros((64, D), jnp.float32)
    h = _project(x, W[0])
    part = _sc_edge(h, eidx, zer)
    x, h = _combine_project(x, part, W[1])
    part = _sc_edge(h, eidx, zer)
    return _combine(x, part)
